# Initial kernel scaffold; baseline (speedup 1.0000x reference)
import jax
import jax.numpy as jnp
from jax.experimental import pallas as pl


def kernel(hidden_states, W, b):
    raise NotImplementedError("write your pallas kernel here")


# speedup vs baseline: 1.7629x; 1.7629x over previous
import functools

import jax
import jax.numpy as jnp
from jax.experimental import pallas as pl

EMBED_DIM = 768
NUM_EXPERTS = 64
TOP_K = 2
LOAD_BALANCE_WEIGHT = 0.01

TOKEN_BLOCK = 2048


def _router_kernel(n_tokens, x_ref, wt_ref, b_ref,
                   idx_ref, pk_ref, probs_ref, usage_ref,
                   loss_ref, ent_ref, conf_ref):
    step = pl.program_id(0)
    nsteps = pl.num_programs(0)

    @pl.when(step == 0)
    def _init():
        usage_ref[...] = jnp.zeros_like(usage_ref)
        ent_ref[...] = jnp.zeros_like(ent_ref)
        conf_ref[...] = jnp.zeros_like(conf_ref)
        loss_ref[...] = jnp.zeros_like(loss_ref)

    x = x_ref[...]
    logits = jnp.dot(x, wt_ref[...],
                     preferred_element_type=jnp.float32) + b_ref[...]
    m = jnp.max(logits, axis=-1, keepdims=True)
    e = jnp.exp(logits - m)
    s = jnp.sum(e, axis=-1, keepdims=True)
    p = e / s
    probs_ref[...] = p

    iota = jax.lax.broadcasted_iota(jnp.int32, p.shape, 1)
    p1 = jnp.max(p, axis=-1, keepdims=True)
    i1 = jnp.min(jnp.where(p == p1, iota, NUM_EXPERTS), axis=-1, keepdims=True)
    pm = jnp.where(iota == i1, -1.0, p)
    p2 = jnp.max(pm, axis=-1, keepdims=True)
    i2 = jnp.min(jnp.where(pm == p2, iota, NUM_EXPERTS), axis=-1, keepdims=True)

    denom = p1 + p2
    p1n = p1 / denom
    p2n = p2 / denom
    idx_ref[...] = jnp.concatenate([i1, i2], axis=1)
    pk_ref[...] = jnp.concatenate([p1n, p2n], axis=1)

    usage_ref[...] += jnp.sum(p, axis=0, keepdims=True)
    ent_ref[...] += jnp.sum(p * jnp.log(p + 1e-8))
    conf_ref[...] += jnp.sum(p1n)

    @pl.when(step == nsteps - 1)
    def _finalize():
        inv_n = 1.0 / n_tokens
        u = usage_ref[...] * inv_n
        usage_ref[...] = u
        t = 1.0 / NUM_EXPERTS
        kl = jnp.sum(t * (jnp.log(t) - jnp.log(u)), keepdims=True) / NUM_EXPERTS
        loss_ref[...] = kl.reshape(1, 1) * LOAD_BALANCE_WEIGHT
        ent_ref[...] = -(ent_ref[...] * inv_n)
        conf_ref[...] = conf_ref[...] * inv_n


@jax.jit
def kernel(hidden_states, W, b):
    B, S, D = hidden_states.shape
    n = B * S
    x = hidden_states.reshape(n, D)
    wt = W.T
    b2 = b.reshape(1, NUM_EXPERTS)
    T = TOKEN_BLOCK
    grid = (n // T,)

    out_shapes = (
        jax.ShapeDtypeStruct((n, TOP_K), jnp.int32),
        jax.ShapeDtypeStruct((n, TOP_K), jnp.float32),
        jax.ShapeDtypeStruct((n, NUM_EXPERTS), jnp.float32),
        jax.ShapeDtypeStruct((1, NUM_EXPERTS), jnp.float32),
        jax.ShapeDtypeStruct((1, 1), jnp.float32),
        jax.ShapeDtypeStruct((1, 1), jnp.float32),
        jax.ShapeDtypeStruct((1, 1), jnp.float32),
    )
    in_specs = [
        pl.BlockSpec((T, D), lambda i: (i, 0)),
        pl.BlockSpec((D, NUM_EXPERTS), lambda i: (0, 0)),
        pl.BlockSpec((1, NUM_EXPERTS), lambda i: (0, 0)),
    ]
    out_specs = (
        pl.BlockSpec((T, TOP_K), lambda i: (i, 0)),
        pl.BlockSpec((T, TOP_K), lambda i: (i, 0)),
        pl.BlockSpec((T, NUM_EXPERTS), lambda i: (i, 0)),
        pl.BlockSpec((1, NUM_EXPERTS), lambda i: (0, 0)),
        pl.BlockSpec((1, 1), lambda i: (0, 0)),
        pl.BlockSpec((1, 1), lambda i: (0, 0)),
        pl.BlockSpec((1, 1), lambda i: (0, 0)),
    )

    idx, pk, probs, usage, loss, ent, conf = pl.pallas_call(
        functools.partial(_router_kernel, float(n)),
        grid=grid,
        in_specs=in_specs,
        out_specs=out_specs,
        out_shape=out_shapes,
    )(x, wt, b2)

    return (idx.reshape(B, S, TOP_K),
            pk.reshape(B, S, TOP_K),
            probs.reshape(B, S, NUM_EXPERTS),
            loss.reshape(()),
            ent.reshape(()),
            usage.reshape(NUM_EXPERTS),
            conf.reshape(()))


# speedup vs baseline: 1.7998x; 1.0209x over previous
import functools

import jax
import jax.numpy as jnp
from jax.experimental import pallas as pl

EMBED_DIM = 768
NUM_EXPERTS = 64
TOP_K = 2
LOAD_BALANCE_WEIGHT = 0.01

TOKEN_BLOCK = 2048


def _router_kernel(n_tokens, x_ref, wt_ref, b_ref,
                   idx_ref, pk_ref, probs_ref, usage_ref,
                   loss_ref, ent_ref, conf_ref):
    step = pl.program_id(0)
    nsteps = pl.num_programs(0)

    @pl.when(step == 0)
    def _init():
        usage_ref[...] = jnp.zeros_like(usage_ref)
        ent_ref[...] = jnp.zeros_like(ent_ref)
        conf_ref[...] = jnp.zeros_like(conf_ref)
        loss_ref[...] = jnp.zeros_like(loss_ref)

    x = x_ref[...]
    logits = jax.lax.dot_general(
        x, wt_ref[...], (((1,), (1,)), ((), ())),
        preferred_element_type=jnp.float32) + b_ref[...]
    m = jnp.max(logits, axis=-1, keepdims=True)
    e = jnp.exp(logits - m)
    s = jnp.sum(e, axis=-1, keepdims=True)
    p = e / s
    probs_ref[...] = p

    iota = jax.lax.broadcasted_iota(jnp.int32, p.shape, 1)
    p1 = jnp.max(p, axis=-1, keepdims=True)
    i1 = jnp.min(jnp.where(p == p1, iota, NUM_EXPERTS), axis=-1, keepdims=True)
    pm = jnp.where(iota == i1, -1.0, p)
    p2 = jnp.max(pm, axis=-1, keepdims=True)
    i2 = jnp.min(jnp.where(pm == p2, iota, NUM_EXPERTS), axis=-1, keepdims=True)

    denom = p1 + p2
    p1n = p1 / denom
    p2n = p2 / denom
    idx_ref[...] = jnp.concatenate([i1, i2], axis=1)
    pk_ref[...] = jnp.concatenate([p1n, p2n], axis=1)

    usage_ref[...] += jnp.sum(p, axis=0, keepdims=True)
    ent_ref[...] += jnp.sum(p * jnp.log(p + 1e-8))
    conf_ref[...] += jnp.sum(p1n)

    @pl.when(step == nsteps - 1)
    def _finalize():
        inv_n = 1.0 / n_tokens
        u = usage_ref[...] * inv_n
        usage_ref[...] = u
        t = 1.0 / NUM_EXPERTS
        kl = jnp.sum(t * (jnp.log(t) - jnp.log(u)), keepdims=True) / NUM_EXPERTS
        loss_ref[...] = kl.reshape(1, 1) * LOAD_BALANCE_WEIGHT
        ent_ref[...] = -(ent_ref[...] * inv_n)
        conf_ref[...] = conf_ref[...] * inv_n


@jax.jit
def kernel(hidden_states, W, b):
    B, S, D = hidden_states.shape
    n = B * S
    x = hidden_states.reshape(n, D)
    b2 = b.reshape(1, NUM_EXPERTS)
    T = TOKEN_BLOCK
    grid = (n // T,)

    out_shapes = (
        jax.ShapeDtypeStruct((n, TOP_K), jnp.int32),
        jax.ShapeDtypeStruct((n, TOP_K), jnp.float32),
        jax.ShapeDtypeStruct((n, NUM_EXPERTS), jnp.float32),
        jax.ShapeDtypeStruct((1, NUM_EXPERTS), jnp.float32),
        jax.ShapeDtypeStruct((1, 1), jnp.float32),
        jax.ShapeDtypeStruct((1, 1), jnp.float32),
        jax.ShapeDtypeStruct((1, 1), jnp.float32),
    )
    in_specs = [
        pl.BlockSpec((T, D), lambda i: (i, 0)),
        pl.BlockSpec((NUM_EXPERTS, D), lambda i: (0, 0)),
        pl.BlockSpec((1, NUM_EXPERTS), lambda i: (0, 0)),
    ]
    out_specs = (
        pl.BlockSpec((T, TOP_K), lambda i: (i, 0)),
        pl.BlockSpec((T, TOP_K), lambda i: (i, 0)),
        pl.BlockSpec((T, NUM_EXPERTS), lambda i: (i, 0)),
        pl.BlockSpec((1, NUM_EXPERTS), lambda i: (0, 0)),
        pl.BlockSpec((1, 1), lambda i: (0, 0)),
        pl.BlockSpec((1, 1), lambda i: (0, 0)),
        pl.BlockSpec((1, 1), lambda i: (0, 0)),
    )

    idx, pk, probs, usage, loss, ent, conf = pl.pallas_call(
        functools.partial(_router_kernel, float(n)),
        grid=grid,
        in_specs=in_specs,
        out_specs=out_specs,
        out_shape=out_shapes,
    )(x, W, b2)

    return (idx.reshape(B, S, TOP_K),
            pk.reshape(B, S, TOP_K),
            probs.reshape(B, S, NUM_EXPERTS),
            loss.reshape(()),
            ent.reshape(()),
            usage.reshape(NUM_EXPERTS),
            conf.reshape(()))


# speedup vs baseline: 1.8858x; 1.0478x over previous
import functools

import jax
import jax.numpy as jnp
from jax.experimental import pallas as pl

EMBED_DIM = 768
NUM_EXPERTS = 64
TOP_K = 2
LOAD_BALANCE_WEIGHT = 0.01

TOKEN_BLOCK = 4096


def _router_kernel(n_tokens, x_ref, wt_ref, b_ref,
                   idx_ref, pk_ref, probs_ref, usage_ref,
                   loss_ref, ent_ref, conf_ref):
    step = pl.program_id(0)
    nsteps = pl.num_programs(0)

    @pl.when(step == 0)
    def _init():
        usage_ref[...] = jnp.zeros_like(usage_ref)
        ent_ref[...] = jnp.zeros_like(ent_ref)
        conf_ref[...] = jnp.zeros_like(conf_ref)
        loss_ref[...] = jnp.zeros_like(loss_ref)

    x = x_ref[...]
    logits = jax.lax.dot_general(
        x, wt_ref[...], (((1,), (1,)), ((), ())),
        preferred_element_type=jnp.float32) + b_ref[...]
    m = jnp.max(logits, axis=-1, keepdims=True)
    e = jnp.exp(logits - m)
    s = jnp.sum(e, axis=-1, keepdims=True)
    p = e / s
    probs_ref[...] = p

    iota = jax.lax.broadcasted_iota(jnp.int32, p.shape, 1)
    p1 = jnp.max(p, axis=-1, keepdims=True)
    i1 = jnp.min(jnp.where(p == p1, iota, NUM_EXPERTS), axis=-1, keepdims=True)
    pm = jnp.where(iota == i1, -1.0, p)
    p2 = jnp.max(pm, axis=-1, keepdims=True)
    i2 = jnp.min(jnp.where(pm == p2, iota, NUM_EXPERTS), axis=-1, keepdims=True)

    denom = p1 + p2
    p1n = p1 / denom
    p2n = p2 / denom
    idx_ref[...] = jnp.concatenate([i1, i2], axis=1)
    pk_ref[...] = jnp.concatenate([p1n, p2n], axis=1)

    usage_ref[...] += jnp.sum(p, axis=0, keepdims=True)
    ent_ref[...] += jnp.sum(p * jnp.log(p + 1e-8))
    conf_ref[...] += jnp.sum(p1n)

    @pl.when(step == nsteps - 1)
    def _finalize():
        inv_n = 1.0 / n_tokens
        u = usage_ref[...] * inv_n
        usage_ref[...] = u
        t = 1.0 / NUM_EXPERTS
        kl = jnp.sum(t * (jnp.log(t) - jnp.log(u)), keepdims=True) / NUM_EXPERTS
        loss_ref[...] = kl.reshape(1, 1) * LOAD_BALANCE_WEIGHT
        ent_ref[...] = -(ent_ref[...] * inv_n)
        conf_ref[...] = conf_ref[...] * inv_n


@jax.jit
def kernel(hidden_states, W, b):
    B, S, D = hidden_states.shape
    n = B * S
    x = hidden_states.reshape(n, D)
    b2 = b.reshape(1, NUM_EXPERTS)
    T = TOKEN_BLOCK
    grid = (n // T,)

    out_shapes = (
        jax.ShapeDtypeStruct((n, TOP_K), jnp.int32),
        jax.ShapeDtypeStruct((n, TOP_K), jnp.float32),
        jax.ShapeDtypeStruct((n, NUM_EXPERTS), jnp.float32),
        jax.ShapeDtypeStruct((1, NUM_EXPERTS), jnp.float32),
        jax.ShapeDtypeStruct((1, 1), jnp.float32),
        jax.ShapeDtypeStruct((1, 1), jnp.float32),
        jax.ShapeDtypeStruct((1, 1), jnp.float32),
    )
    in_specs = [
        pl.BlockSpec((T, D), lambda i: (i, 0)),
        pl.BlockSpec((NUM_EXPERTS, D), lambda i: (0, 0)),
        pl.BlockSpec((1, NUM_EXPERTS), lambda i: (0, 0)),
    ]
    out_specs = (
        pl.BlockSpec((T, TOP_K), lambda i: (i, 0)),
        pl.BlockSpec((T, TOP_K), lambda i: (i, 0)),
        pl.BlockSpec((T, NUM_EXPERTS), lambda i: (i, 0)),
        pl.BlockSpec((1, NUM_EXPERTS), lambda i: (0, 0)),
        pl.BlockSpec((1, 1), lambda i: (0, 0)),
        pl.BlockSpec((1, 1), lambda i: (0, 0)),
        pl.BlockSpec((1, 1), lambda i: (0, 0)),
    )

    idx, pk, probs, usage, loss, ent, conf = pl.pallas_call(
        functools.partial(_router_kernel, float(n)),
        grid=grid,
        in_specs=in_specs,
        out_specs=out_specs,
        out_shape=out_shapes,
    )(x, W, b2)

    return (idx.reshape(B, S, TOP_K),
            pk.reshape(B, S, TOP_K),
            probs.reshape(B, S, NUM_EXPERTS),
            loss.reshape(()),
            ent.reshape(()),
            usage.reshape(NUM_EXPERTS),
            conf.reshape(()))


# speedup vs baseline: 1.8890x; 1.0017x over previous
import functools

import jax
import jax.numpy as jnp
from jax.experimental import pallas as pl

EMBED_DIM = 768
NUM_EXPERTS = 64
TOP_K = 2
LOAD_BALANCE_WEIGHT = 0.01

TOKEN_BLOCK = 4096
STREAMS = 4


def _router_kernel(n_tokens, x0_ref, x1_ref, x2_ref, x3_ref, wt_ref, b_ref,
                   idx_ref, pk_ref, probs_ref, usage_ref,
                   loss_ref, ent_ref, conf_ref):
    step = pl.program_id(0)
    nsteps = pl.num_programs(0)

    @pl.when(step == 0)
    def _init():
        usage_ref[...] = jnp.zeros_like(usage_ref)
        ent_ref[...] = jnp.zeros_like(ent_ref)
        conf_ref[...] = jnp.zeros_like(conf_ref)
        loss_ref[...] = jnp.zeros_like(loss_ref)

    w = wt_ref[...]
    logits = jnp.concatenate(
        [jax.lax.dot_general(xr[...], w, (((1,), (1,)), ((), ())),
                             preferred_element_type=jnp.float32)
         for xr in (x0_ref, x1_ref, x2_ref, x3_ref)],
        axis=0) + b_ref[...]
    m = jnp.max(logits, axis=-1, keepdims=True)
    e = jnp.exp(logits - m)
    s = jnp.sum(e, axis=-1, keepdims=True)
    p = e / s
    probs_ref[...] = p

    iota = jax.lax.broadcasted_iota(jnp.int32, p.shape, 1)
    p1 = jnp.max(p, axis=-1, keepdims=True)
    i1 = jnp.min(jnp.where(p == p1, iota, NUM_EXPERTS), axis=-1, keepdims=True)
    pm = jnp.where(iota == i1, -1.0, p)
    p2 = jnp.max(pm, axis=-1, keepdims=True)
    i2 = jnp.min(jnp.where(pm == p2, iota, NUM_EXPERTS), axis=-1, keepdims=True)

    denom = p1 + p2
    p1n = p1 / denom
    p2n = p2 / denom
    idx_ref[...] = jnp.concatenate([i1, i2], axis=1)
    pk_ref[...] = jnp.concatenate([p1n, p2n], axis=1)

    usage_ref[...] += jnp.sum(p, axis=0, keepdims=True)
    ent_ref[...] += jnp.sum(p * jnp.log(p + 1e-8))
    conf_ref[...] += jnp.sum(p1n)

    @pl.when(step == nsteps - 1)
    def _finalize():
        inv_n = 1.0 / n_tokens
        u = usage_ref[...] * inv_n
        usage_ref[...] = u
        t = 1.0 / NUM_EXPERTS
        kl = jnp.sum(t * (jnp.log(t) - jnp.log(u)), keepdims=True) / NUM_EXPERTS
        loss_ref[...] = kl.reshape(1, 1) * LOAD_BALANCE_WEIGHT
        ent_ref[...] = -(ent_ref[...] * inv_n)
        conf_ref[...] = conf_ref[...] * inv_n


@jax.jit
def kernel(hidden_states, W, b):
    B, S, D = hidden_states.shape
    n = B * S
    x = hidden_states.reshape(n, D)
    b2 = b.reshape(1, NUM_EXPERTS)
    T = TOKEN_BLOCK
    grid = (n // T,)

    out_shapes = (
        jax.ShapeDtypeStruct((n, TOP_K), jnp.int32),
        jax.ShapeDtypeStruct((n, TOP_K), jnp.float32),
        jax.ShapeDtypeStruct((n, NUM_EXPERTS), jnp.float32),
        jax.ShapeDtypeStruct((1, NUM_EXPERTS), jnp.float32),
        jax.ShapeDtypeStruct((1, 1), jnp.float32),
        jax.ShapeDtypeStruct((1, 1), jnp.float32),
        jax.ShapeDtypeStruct((1, 1), jnp.float32),
    )
    Ts = T // STREAMS
    in_specs = [
        pl.BlockSpec((Ts, D), (lambda i, j=j: (i * STREAMS + j, 0)))
        for j in range(STREAMS)
    ] + [
        pl.BlockSpec((NUM_EXPERTS, D), lambda i: (0, 0)),
        pl.BlockSpec((1, NUM_EXPERTS), lambda i: (0, 0)),
    ]
    out_specs = (
        pl.BlockSpec((T, TOP_K), lambda i: (i, 0)),
        pl.BlockSpec((T, TOP_K), lambda i: (i, 0)),
        pl.BlockSpec((T, NUM_EXPERTS), lambda i: (i, 0)),
        pl.BlockSpec((1, NUM_EXPERTS), lambda i: (0, 0)),
        pl.BlockSpec((1, 1), lambda i: (0, 0)),
        pl.BlockSpec((1, 1), lambda i: (0, 0)),
        pl.BlockSpec((1, 1), lambda i: (0, 0)),
    )

    idx, pk, probs, usage, loss, ent, conf = pl.pallas_call(
        functools.partial(_router_kernel, float(n)),
        grid=grid,
        in_specs=in_specs,
        out_specs=out_specs,
        out_shape=out_shapes,
    )(x, x, x, x, W, b2)

    return (idx.reshape(B, S, TOP_K),
            pk.reshape(B, S, TOP_K),
            probs.reshape(B, S, NUM_EXPERTS),
            loss.reshape(()),
            ent.reshape(()),
            usage.reshape(NUM_EXPERTS),
            conf.reshape(()))
